# baseline (speedup 1.0000x reference)
import functools

import jax
import jax.numpy as jnp
from jax import lax
from jax.experimental import pallas as pl
from jax.experimental.pallas import tpu as pltpu
from jax.experimental.pallas import tpu_sc as plsc

M = 100000
D = 64
B = 16384
M2 = M // 2
D2 = 2 * D
NC, NS = 2, 16
NW = NC * NS
BPW = B // NW
BPT = B // NS
NBLK = 6
BLK = 8336
SCRAP = 512
NCH = BPT // 128

_mesh = plsc.VectorSubcoreMesh(core_axis_name="c", subcore_axis_name="s")


def _bcast(v, lane):
    return lax.gather(
        v, jnp.full((16, 1), lane, jnp.int32),
        lax.GatherDimensionNumbers(offset_dims=(), collapsed_slice_dims=(0,),
                                   start_index_map=(0,)),
        slice_sizes=(1,), mode=lax.GatherScatterMode.PROMISE_IN_BOUNDS)


def _bcast0(v):
    return _bcast(v, 0)


def _bcast15(v):
    return _bcast(v, 15)


@functools.partial(
    pl.kernel, mesh=_mesh,
    out_type=jax.ShapeDtypeStruct((B, D2), jnp.float32),
    scratch_types=[
        pltpu.VMEM((BPW // 128, 128), jnp.int32),
        pltpu.VMEM((BPW, D2), jnp.float32),
    ],
)
def _sc_gather(bank_ref, idx_hbm, old2_hbm, idx_v, rows_v):
    wid = lax.axis_index("s") * NC + lax.axis_index("c")
    pltpu.sync_copy(idx_hbm.at[wid], idx_v)
    for j in range(BPW // 128):
        pltpu.sync_copy(bank_ref.at[idx_v.at[j]],
                        rows_v.at[pl.ds(j * 128, 128)])
    pltpu.sync_copy(rows_v, old2_hbm.at[pl.ds(wid * BPW, BPW)])


@functools.partial(
    pl.kernel, mesh=_mesh,
    out_type=(),
    compiler_params=pltpu.CompilerParams(needs_layout_passes=False),
    scratch_types=[
        pltpu.VMEM_SHARED((BLK + SCRAP, D2), jnp.float32),
        pltpu.VMEM((NCH, 128), jnp.int32),
        pltpu.VMEM((NCH, 128), jnp.int32),
        pltpu.VMEM((NCH, 128), jnp.int32),
        pltpu.VMEM((NCH, 128), jnp.int32),
        pltpu.VMEM((128, D2), jnp.float32),
    ],
)
def _sc_scatter(bank_ref, delta2_hbm, idx_hbm,
                acc_sh, idx_v, selj_v, selg_v, sell_v, buf_v):
    cid = lax.axis_index("c")
    sid = lax.axis_index("s")
    pltpu.sync_copy(idx_hbm.at[sid], idx_v)

    def block_body(b):
        blk = cid * (NBLK // NC) + b
        start = jnp.minimum(blk * BLK, M2 - BLK)
        lo = blk * BLK
        hi = jnp.minimum(lo + BLK, M2)
        off = jnp.zeros((16,), jnp.int32)
        ones = jnp.ones((16,), jnp.int32)
        for i in range(BPT // 16):
            iv = idx_v[i // 8, pl.ds((i % 8) * 16, 16)]
            inb = (iv >= lo) & (iv < hi)
            inc = plsc.cumsum(jnp.where(inb, 1, 0))
            pos = jnp.maximum(off + inc - 1, 0)
            row = lax.shift_right_logical(pos, 7)
            col = pos & 127
            jrow = lax.iota(jnp.int32, 16) + (sid * BPT + i * 16)
            plsc.store_scatter(selj_v, [row, col], jrow, mask=inb)
            plsc.store_scatter(selg_v, [row, col], iv, mask=inb)
            plsc.store_scatter(sell_v, [row, col], iv - start, mask=inb)
            off = off + _bcast15(inc)
        fg = _bcast0(selg_v[0, pl.ds(0, 16)])
        fl = _bcast0(sell_v[0, pl.ds(0, 16)])
        zrow = B + (lax.iota(jnp.int32, 16) + sid * 128)
        for i in range(BPT // 16):
            sl = pl.ds((i % 8) * 16, 16)
            lanepos = lax.iota(jnp.int32, 16) + i * 16
            pad = lanepos >= off
            r = i // 8
            selj_v[r, sl] = jnp.where(pad, zrow, selj_v[r, sl])
            selg_v[r, sl] = jnp.where(pad, fg, selg_v[r, sl])
            sell_v[r, sl] = jnp.where(pad, fl, sell_v[r, sl])
        for c in range(NCH):
            @pl.when(jnp.any(off > c * 128))
            def _():
                pltpu.sync_copy(bank_ref.at[selg_v.at[c]], buf_v)
                pltpu.sync_copy(buf_v, acc_sh.at[sell_v.at[c]])
        plsc.subcore_barrier()
        for c in range(NCH):
            @pl.when(jnp.any(off > c * 128))
            def _():
                pltpu.sync_copy(delta2_hbm.at[selj_v.at[c]], buf_v)
                pltpu.sync_copy(buf_v, acc_sh.at[sell_v.at[c]], add=True)
        plsc.subcore_barrier()
        for c in range(NCH):
            @pl.when(jnp.any(off > c * 128))
            def _():
                pltpu.sync_copy(acc_sh.at[sell_v.at[c]], buf_v)
                pltpu.sync_copy(buf_v, bank_ref.at[selg_v.at[c]])
        plsc.subcore_barrier()

    for _b in range(NBLK // NC):
        block_body(jnp.int32(_b))


def _delta_body(old2_ref, val_ref, par_ref, wg_ref, bg_ref, wv_ref, out_ref):
    @pl.when(pl.program_id(0) == B // _TCB)
    def _():
        out_ref[...] = jnp.zeros_like(out_ref)
    @pl.when(pl.program_id(0) < B // _TCB)
    def _():
        _delta_compute(old2_ref, val_ref, par_ref, wg_ref, bg_ref, wv_ref,
                       out_ref)


def _delta_compute(old2_ref, val_ref, par_ref, wg_ref, bg_ref, wv_ref, out_ref):
    p = par_ref[0]
    old2 = old2_ref[...]
    old = old2[:, :D] * (1.0 - p) + old2[:, D:] * p
    v = val_ref[...]
    pre = jnp.dot(old + v, wg_ref[...], preferred_element_type=jnp.float32)
    gate = jax.nn.sigmoid(pre + bg_ref[...])
    upd = jnp.tanh(jnp.dot(v, wv_ref[...], preferred_element_type=jnp.float32))
    d = gate * (upd - old)
    out_ref[...] = jnp.concatenate([d * (1.0 - p), d * p], axis=1)


_TCB = 2048


def _tc_delta(old2, values, par, W_gate, b_gate, W_val):
    nb = B // _TCB
    return pl.pallas_call(
        _delta_body,
        grid=(nb + 1,),
        in_specs=[
            pl.BlockSpec((_TCB, D2), lambda i: (jnp.minimum(i, nb - 1), 0)),
            pl.BlockSpec((_TCB, D), lambda i: (jnp.minimum(i, nb - 1), 0)),
            pl.BlockSpec((1, _TCB, 1), lambda i: (jnp.minimum(i, nb - 1), 0, 0)),
            pl.BlockSpec((D, D), lambda i: (0, 0)),
            pl.BlockSpec((1, D), lambda i: (0, 0)),
            pl.BlockSpec((D, D), lambda i: (0, 0)),
        ],
        out_specs=pl.BlockSpec((_TCB, D2), lambda i: (i, 0)),
        out_shape=jax.ShapeDtypeStruct((B + _TCB, D2), jnp.float32),
    )(old2, values, par, W_gate, b_gate, W_val)


def kernel(memory, indices, values, W_gate, b_gate, W_val):
    idx = indices.astype(jnp.int32)
    idxp = idx >> 1
    par = (idx & 1).astype(jnp.float32).reshape(B // _TCB, _TCB, 1)
    bank = jax.new_ref(memory.reshape(M2, D2))
    old2 = _sc_gather(bank, idxp.reshape(NW, BPW // 128, 128))
    delta2 = _tc_delta(old2, values, par, W_gate, b_gate.reshape(1, D), W_val)
    _sc_scatter(bank, delta2, idxp.reshape(NS, BPT // 128, 128))
    return bank[...].reshape(M, D)


# speedup vs baseline: 1.1017x; 1.1017x over previous
import functools

import jax
import jax.numpy as jnp
from jax import lax
from jax.experimental import pallas as pl
from jax.experimental.pallas import tpu as pltpu
from jax.experimental.pallas import tpu_sc as plsc

M = 100000
D = 64
B = 16384
M2 = M // 2
D2 = 2 * D
NC, NS = 2, 16
NW = NC * NS
BPW = B // NW
BPT = B // NS
NBLK = 6
BLK = 8336
SCRAP = 512
CHUNK = 528
NCH = BPT // 128

_mesh = plsc.VectorSubcoreMesh(core_axis_name="c", subcore_axis_name="s")
_params = pltpu.CompilerParams(needs_layout_passes=False)


def _bcast15(v):
    return lax.gather(
        v, jnp.full((16, 1), 15, jnp.int32),
        lax.GatherDimensionNumbers(offset_dims=(), collapsed_slice_dims=(0,),
                                   start_index_map=(0,)),
        slice_sizes=(1,), mode=lax.GatherScatterMode.PROMISE_IN_BOUNDS)


def _load_idx_2d(idx_hbm, base, n, idx1_v, idx2_v):
    pltpu.sync_copy(idx_hbm.at[pl.ds(base, n)], idx1_v)
    for i in range(n // 16):
        idx2_v[i // 8, pl.ds((i % 8) * 16, 16)] = idx1_v[pl.ds(i * 16, 16)]


@functools.partial(
    pl.kernel, mesh=_mesh,
    out_type=jax.ShapeDtypeStruct((B, D2), jnp.float32),
    compiler_params=_params,
    scratch_types=[
        pltpu.VMEM((BPW,), jnp.int32),
        pltpu.VMEM((BPW // 128, 128), jnp.int32),
        pltpu.VMEM((BPW, D2), jnp.float32),
    ],
)
def _sc_gather(mem2_hbm, idx_hbm, old2_hbm, idx1_v, idx2_v, rows_v):
    wid = lax.axis_index("s") * NC + lax.axis_index("c")
    _load_idx_2d(idx_hbm, wid * BPW, BPW, idx1_v, idx2_v)
    for j in range(BPW // 128):
        pltpu.sync_copy(mem2_hbm.at[idx2_v.at[j]],
                        rows_v.at[pl.ds(j * 128, 128)])
    pltpu.sync_copy(rows_v, old2_hbm.at[pl.ds(wid * BPW, BPW)])


@functools.partial(
    pl.kernel, mesh=_mesh,
    out_type=jax.ShapeDtypeStruct((M2, D2), jnp.float32),
    compiler_params=_params,
    scratch_types=[
        pltpu.VMEM_SHARED((BLK + SCRAP, D2), jnp.float32),
        pltpu.VMEM((BPT,), jnp.int32),
        pltpu.VMEM((NCH, 128), jnp.int32),
        pltpu.VMEM((NCH, 128), jnp.int32),
        pltpu.VMEM((NCH, 128), jnp.int32),
        pltpu.VMEM((128, D2), jnp.float32),
    ],
)
def _sc_scatter(mem2_hbm, delta2_hbm, idx_hbm, out2_hbm,
                acc_sh, idx1_v, idx_v, selj_v, sell_v, buf_v):
    cid = lax.axis_index("c")
    sid = lax.axis_index("s")
    _load_idx_2d(idx_hbm, sid * BPT, BPT, idx1_v, idx_v)
    cs = jnp.minimum(sid * CHUNK, BLK - CHUNK)
    for b in range(NBLK // NC):
        blk = cid * (NBLK // NC) + b
        start = jnp.minimum(blk * BLK, M2 - BLK)
        pltpu.sync_copy(mem2_hbm.at[pl.ds(start + cs, CHUNK)],
                        acc_sh.at[pl.ds(cs, CHUNK)])
        for i in range(BPT // 16):
            lane = lax.iota(jnp.int32, 16) + i * 16
            sl = pl.ds((i % 8) * 16, 16)
            selj_v[i // 8, sl] = (lane + sid * 64) & (B - 1)
            sell_v[i // 8, sl] = BLK + ((lane + sid * 32) & (SCRAP - 1))
        off = jnp.zeros((16,), jnp.int32)
        for i in range(BPT // 16):
            iv = idx_v[i // 8, pl.ds((i % 8) * 16, 16)]
            inb = (iv >= start) & (iv < start + BLK)
            inc = plsc.cumsum(jnp.where(inb, 1, 0))
            pos = jnp.maximum(off + inc - 1, 0)
            row = lax.shift_right_logical(pos, 7)
            col = pos & 127
            jrow = lax.iota(jnp.int32, 16) + (sid * BPT + i * 16)
            plsc.store_scatter(selj_v, [row, col], jrow, mask=inb)
            plsc.store_scatter(sell_v, [row, col], iv - start, mask=inb)
            off = off + _bcast15(inc)
        plsc.subcore_barrier()
        for c in range(NCH):
            @pl.when(jnp.any(off > c * 128))
            def _():
                pltpu.sync_copy(delta2_hbm.at[selj_v.at[c]], buf_v)
                pltpu.sync_copy(buf_v, acc_sh.at[sell_v.at[c]], add=True)
        plsc.subcore_barrier()
        pltpu.sync_copy(acc_sh.at[pl.ds(cs, CHUNK)],
                        out2_hbm.at[pl.ds(start + cs, CHUNK)])
        plsc.subcore_barrier()


def _delta_body(old2_ref, val_ref, par_ref, wg_ref, bg_ref, wv_ref, out_ref):
    p = par_ref[0]
    old2 = old2_ref[...]
    old = old2[:, :D] * (1.0 - p) + old2[:, D:] * p
    v = val_ref[...]
    pre = jnp.dot(old + v, wg_ref[...], preferred_element_type=jnp.float32)
    gate = jax.nn.sigmoid(pre + bg_ref[...])
    upd = jnp.tanh(jnp.dot(v, wv_ref[...], preferred_element_type=jnp.float32))
    d = gate * (upd - old)
    out_ref[...] = jnp.concatenate([d * (1.0 - p), d * p], axis=1)


_TCB = 4096


def _tc_delta(old2, values, par, W_gate, b_gate, W_val):
    return pl.pallas_call(
        _delta_body,
        grid=(B // _TCB,),
        in_specs=[
            pl.BlockSpec((_TCB, D2), lambda i: (i, 0)),
            pl.BlockSpec((_TCB, D), lambda i: (i, 0)),
            pl.BlockSpec((1, _TCB, 1), lambda i: (i, 0, 0)),
            pl.BlockSpec((D, D), lambda i: (0, 0)),
            pl.BlockSpec((1, D), lambda i: (0, 0)),
            pl.BlockSpec((D, D), lambda i: (0, 0)),
        ],
        out_specs=pl.BlockSpec((_TCB, D2), lambda i: (i, 0)),
        out_shape=jax.ShapeDtypeStruct((B, D2), jnp.float32),
    )(old2, values, par, W_gate, b_gate, W_val)


def kernel(memory, indices, values, W_gate, b_gate, W_val):
    idx = indices.astype(jnp.int32)
    idxp = idx >> 1
    par = (idx & 1).astype(jnp.float32).reshape(B // _TCB, _TCB, 1)
    mem2 = memory.reshape(M2, D2)
    old2 = _sc_gather(mem2, idxp)
    delta2 = _tc_delta(old2, values, par, W_gate, b_gate.reshape(1, D), W_val)
    out2 = _sc_scatter(mem2, delta2, idxp)
    return out2.reshape(M, D)


# speedup vs baseline: 1.1036x; 1.0017x over previous
import functools

import jax
import jax.numpy as jnp
from jax import lax
from jax.experimental import pallas as pl
from jax.experimental.pallas import tpu as pltpu
from jax.experimental.pallas import tpu_sc as plsc

M = 100000
D = 64
B = 16384
M2 = M // 2
D2 = 2 * D
NC, NS = 2, 16
NW = NC * NS
BPW = B // NW
BPT = B // NS
NBLK = 6
BLK = 8336
SCRAP = 512
CHUNK = 528
NCH = BPT // 128

_mesh = plsc.VectorSubcoreMesh(core_axis_name="c", subcore_axis_name="s")
_params = pltpu.CompilerParams(needs_layout_passes=False,
                               use_tc_tiling_on_sc=False)


def _bcast15(v):
    return lax.gather(
        v, jnp.full((16, 1), 15, jnp.int32),
        lax.GatherDimensionNumbers(offset_dims=(), collapsed_slice_dims=(0,),
                                   start_index_map=(0,)),
        slice_sizes=(1,), mode=lax.GatherScatterMode.PROMISE_IN_BOUNDS)


def _load_idx_2d(idx_hbm, base, n, idx1_v, idx2_v):
    pltpu.sync_copy(idx_hbm.at[pl.ds(base, n)], idx1_v)
    for i in range(n // 16):
        idx2_v[i // 8, pl.ds((i % 8) * 16, 16)] = idx1_v[pl.ds(i * 16, 16)]


@functools.partial(
    pl.kernel, mesh=_mesh,
    out_type=jax.ShapeDtypeStruct((B, D2), jnp.float32),
    compiler_params=_params,
    scratch_types=[
        pltpu.VMEM((BPW,), jnp.int32),
        pltpu.VMEM((BPW // 128, 128), jnp.int32),
        pltpu.VMEM((BPW, D2), jnp.float32),
    ],
)
def _sc_gather(mem2_hbm, idx_hbm, old2_hbm, idx1_v, idx2_v, rows_v):
    wid = lax.axis_index("s") * NC + lax.axis_index("c")
    _load_idx_2d(idx_hbm, wid * BPW, BPW, idx1_v, idx2_v)
    for j in range(BPW // 128):
        pltpu.sync_copy(mem2_hbm.at[idx2_v.at[j]],
                        rows_v.at[pl.ds(j * 128, 128)])
    pltpu.sync_copy(rows_v, old2_hbm.at[pl.ds(wid * BPW, BPW)])


@functools.partial(
    pl.kernel, mesh=_mesh,
    out_type=jax.ShapeDtypeStruct((M2, D2), jnp.float32),
    compiler_params=_params,
    scratch_types=[
        pltpu.VMEM_SHARED((BLK + SCRAP, D2), jnp.float32),
        pltpu.VMEM((BPT,), jnp.int32),
        pltpu.VMEM((NCH, 128), jnp.int32),
        pltpu.VMEM((NCH, 128), jnp.int32),
        pltpu.VMEM((NCH, 128), jnp.int32),
        pltpu.VMEM((128, D2), jnp.float32),
    ],
)
def _sc_scatter(mem2_hbm, delta2_hbm, idx_hbm, out2_hbm,
                acc_sh, idx1_v, idx_v, selj_v, sell_v, buf_v):
    cid = lax.axis_index("c")
    sid = lax.axis_index("s")
    _load_idx_2d(idx_hbm, sid * BPT, BPT, idx1_v, idx_v)
    cs = jnp.minimum(sid * CHUNK, BLK - CHUNK)
    for b in range(NBLK // NC):
        blk = cid * (NBLK // NC) + b
        start = jnp.minimum(blk * BLK, M2 - BLK)
        pltpu.sync_copy(mem2_hbm.at[pl.ds(start + cs, CHUNK)],
                        acc_sh.at[pl.ds(cs, CHUNK)])
        for i in range(BPT // 16):
            lane = lax.iota(jnp.int32, 16) + i * 16
            sl = pl.ds((i % 8) * 16, 16)
            selj_v[i // 8, sl] = (lane + sid * 64) & (B - 1)
            sell_v[i // 8, sl] = BLK + ((lane + sid * 32) & (SCRAP - 1))
        off = jnp.zeros((16,), jnp.int32)
        for i in range(BPT // 16):
            iv = idx_v[i // 8, pl.ds((i % 8) * 16, 16)]
            inb = (iv >= start) & (iv < start + BLK)
            inc = plsc.cumsum(jnp.where(inb, 1, 0))
            pos = jnp.maximum(off + inc - 1, 0)
            row = lax.shift_right_logical(pos, 7)
            col = pos & 127
            jrow = lax.iota(jnp.int32, 16) + (sid * BPT + i * 16)
            plsc.store_scatter(selj_v, [row, col], jrow, mask=inb)
            plsc.store_scatter(sell_v, [row, col], iv - start, mask=inb)
            off = off + _bcast15(inc)
        plsc.subcore_barrier()
        for c in range(NCH):
            @pl.when(jnp.any(off > c * 128))
            def _():
                pltpu.sync_copy(delta2_hbm.at[selj_v.at[c]], buf_v)
                pltpu.sync_copy(buf_v, acc_sh.at[sell_v.at[c]], add=True)
        plsc.subcore_barrier()
        pltpu.sync_copy(acc_sh.at[pl.ds(cs, CHUNK)],
                        out2_hbm.at[pl.ds(start + cs, CHUNK)])
        plsc.subcore_barrier()


def _delta_body(old2_ref, val_ref, par_ref, wg_ref, bg_ref, wv_ref, out_ref):
    p = par_ref[0]
    old2 = old2_ref[...]
    old = old2[:, :D] * (1.0 - p) + old2[:, D:] * p
    v = val_ref[...]
    pre = jnp.dot(old + v, wg_ref[...], preferred_element_type=jnp.float32)
    gate = jax.nn.sigmoid(pre + bg_ref[...])
    upd = jnp.tanh(jnp.dot(v, wv_ref[...], preferred_element_type=jnp.float32))
    d = gate * (upd - old)
    out_ref[...] = jnp.concatenate([d * (1.0 - p), d * p], axis=1)


_TCB = 4096


def _tc_delta(old2, values, par, W_gate, b_gate, W_val):
    return pl.pallas_call(
        _delta_body,
        grid=(B // _TCB,),
        in_specs=[
            pl.BlockSpec((_TCB, D2), lambda i: (i, 0)),
            pl.BlockSpec((_TCB, D), lambda i: (i, 0)),
            pl.BlockSpec((1, _TCB, 1), lambda i: (i, 0, 0)),
            pl.BlockSpec((D, D), lambda i: (0, 0)),
            pl.BlockSpec((1, D), lambda i: (0, 0)),
            pl.BlockSpec((D, D), lambda i: (0, 0)),
        ],
        out_specs=pl.BlockSpec((_TCB, D2), lambda i: (i, 0)),
        out_shape=jax.ShapeDtypeStruct((B, D2), jnp.float32),
    )(old2, values, par, W_gate, b_gate, W_val)


def kernel(memory, indices, values, W_gate, b_gate, W_val):
    idx = indices.astype(jnp.int32)
    idxp = idx >> 1
    par = (idx & 1).astype(jnp.float32).reshape(B // _TCB, _TCB, 1)
    mem2 = memory.reshape(M2, D2)
    old2 = _sc_gather(mem2, idxp)
    delta2 = _tc_delta(old2, values, par, W_gate, b_gate.reshape(1, D), W_val)
    out2 = _sc_scatter(mem2, delta2, idxp)
    return out2.reshape(M, D)


# speedup vs baseline: 1.1225x; 1.0171x over previous
import functools

import jax
import jax.numpy as jnp
from jax import lax
from jax.experimental import pallas as pl
from jax.experimental.pallas import tpu as pltpu
from jax.experimental.pallas import tpu_sc as plsc

M = 100000
D = 64
B = 16384
NC, NS = 2, 16
NW = NC * NS
BPW = B // NW
BPT = B // NS
NBLK = 6
BLK = 16672
SCRAP = 512
CHUNK = 1048
NCH = BPT // 128

_mesh = plsc.VectorSubcoreMesh(core_axis_name="c", subcore_axis_name="s")
_params = pltpu.CompilerParams(needs_layout_passes=False,
                               use_tc_tiling_on_sc=False)


def _bcast15(v):
    return lax.gather(
        v, jnp.full((16, 1), 15, jnp.int32),
        lax.GatherDimensionNumbers(offset_dims=(), collapsed_slice_dims=(0,),
                                   start_index_map=(0,)),
        slice_sizes=(1,), mode=lax.GatherScatterMode.PROMISE_IN_BOUNDS)


def _load_idx_2d(idx_hbm, base, n, idx1_v, idx2_v):
    pltpu.sync_copy(idx_hbm.at[pl.ds(base, n)], idx1_v)
    for i in range(n // 16):
        idx2_v[i // 8, pl.ds((i % 8) * 16, 16)] = idx1_v[pl.ds(i * 16, 16)]


@functools.partial(
    pl.kernel, mesh=_mesh,
    out_type=jax.ShapeDtypeStruct((B, D), jnp.float32),
    compiler_params=_params,
    scratch_types=[
        pltpu.VMEM((BPW,), jnp.int32),
        pltpu.VMEM((BPW // 128, 128), jnp.int32),
        pltpu.VMEM((BPW, D), jnp.float32),
    ],
)
def _sc_gather(mem_hbm, idx_hbm, old_hbm, idx1_v, idx2_v, rows_v):
    wid = lax.axis_index("s") * NC + lax.axis_index("c")
    _load_idx_2d(idx_hbm, wid * BPW, BPW, idx1_v, idx2_v)
    for j in range(BPW // 128):
        pltpu.sync_copy(mem_hbm.at[idx2_v.at[j]],
                        rows_v.at[pl.ds(j * 128, 128)])
    pltpu.sync_copy(rows_v, old_hbm.at[pl.ds(wid * BPW, BPW)])


@functools.partial(
    pl.kernel, mesh=_mesh,
    out_type=jax.ShapeDtypeStruct((M, D), jnp.float32),
    compiler_params=_params,
    scratch_types=[
        pltpu.VMEM_SHARED((BLK + SCRAP, D), jnp.float32),
        pltpu.VMEM((BPT,), jnp.int32),
        pltpu.VMEM((NCH, 128), jnp.int32),
        pltpu.VMEM((NCH, 128), jnp.int32),
        pltpu.VMEM((NCH, 128), jnp.int32),
        pltpu.VMEM((128, D), jnp.float32),
    ],
)
def _sc_scatter(mem_hbm, delta_hbm, idx_hbm, out_hbm,
                acc_sh, idx1_v, idx_v, selj_v, sell_v, buf_v):
    cid = lax.axis_index("c")
    sid = lax.axis_index("s")
    _load_idx_2d(idx_hbm, sid * BPT, BPT, idx1_v, idx_v)
    cs = jnp.minimum(sid * CHUNK, BLK - CHUNK)
    for b in range(NBLK // NC):
        blk = cid * (NBLK // NC) + b
        start = jnp.minimum(blk * BLK, M - BLK)
        pltpu.sync_copy(mem_hbm.at[pl.ds(start + cs, CHUNK)],
                        acc_sh.at[pl.ds(cs, CHUNK)])
        for i in range(BPT // 16):
            lane = lax.iota(jnp.int32, 16) + i * 16
            sl = pl.ds((i % 8) * 16, 16)
            selj_v[i // 8, sl] = (lane + sid * 64) & (B - 1)
            sell_v[i // 8, sl] = BLK + ((lane + sid * 32) & (SCRAP - 1))
        off = jnp.zeros((16,), jnp.int32)
        for i in range(BPT // 16):
            iv = idx_v[i // 8, pl.ds((i % 8) * 16, 16)]
            inb = (iv >= start) & (iv < start + BLK)
            inc = plsc.cumsum(jnp.where(inb, 1, 0))
            pos = jnp.maximum(off + inc - 1, 0)
            row = lax.shift_right_logical(pos, 7)
            col = pos & 127
            jrow = lax.iota(jnp.int32, 16) + (sid * BPT + i * 16)
            plsc.store_scatter(selj_v, [row, col], jrow, mask=inb)
            plsc.store_scatter(sell_v, [row, col], iv - start, mask=inb)
            off = off + _bcast15(inc)
        plsc.subcore_barrier()
        for c in range(NCH):
            @pl.when(jnp.any(off > c * 128))
            def _():
                pltpu.sync_copy(delta_hbm.at[selj_v.at[c]], buf_v)
                pltpu.sync_copy(buf_v, acc_sh.at[sell_v.at[c]], add=True)
        plsc.subcore_barrier()
        pltpu.sync_copy(acc_sh.at[pl.ds(cs, CHUNK)],
                        out_hbm.at[pl.ds(start + cs, CHUNK)])
        plsc.subcore_barrier()


def _delta_body(old_ref, val_ref, wg_ref, bg_ref, wv_ref, out_ref):
    old = old_ref[...]
    v = val_ref[...]
    pre = jnp.dot(old + v, wg_ref[...], preferred_element_type=jnp.float32)
    gate = jax.nn.sigmoid(pre + bg_ref[...])
    upd = jnp.tanh(jnp.dot(v, wv_ref[...], preferred_element_type=jnp.float32))
    out_ref[...] = gate * (upd - old)


_TCB = 4096


def _tc_delta(old, values, W_gate, b_gate, W_val):
    return pl.pallas_call(
        _delta_body,
        grid=(B // _TCB,),
        in_specs=[
            pl.BlockSpec((_TCB, D), lambda i: (i, 0)),
            pl.BlockSpec((_TCB, D), lambda i: (i, 0)),
            pl.BlockSpec((D, D), lambda i: (0, 0)),
            pl.BlockSpec((1, D), lambda i: (0, 0)),
            pl.BlockSpec((D, D), lambda i: (0, 0)),
        ],
        out_specs=pl.BlockSpec((_TCB, D), lambda i: (i, 0)),
        out_shape=jax.ShapeDtypeStruct((B, D), jnp.float32),
    )(old, values, W_gate, b_gate, W_val)


def kernel(memory, indices, values, W_gate, b_gate, W_val):
    idx = indices.astype(jnp.int32)
    old = _sc_gather(memory, idx)
    delta = _tc_delta(old, values, W_gate, b_gate.reshape(1, D), W_val)
    return _sc_scatter(memory, delta, idx)


# speedup vs baseline: 1.1591x; 1.0326x over previous
import functools

import jax
import jax.numpy as jnp
from jax import lax
from jax.experimental import pallas as pl
from jax.experimental.pallas import tpu as pltpu
from jax.experimental.pallas import tpu_sc as plsc

M = 100000
D = 64
B = 16384
NC, NS = 2, 16
NW = NC * NS
BPW = B // NW
BPT = B // NS
NBLK = 4
BLK = 25008
SCRAP = 512
CHUNK = 1568
NCH = BPT // 128

_mesh = plsc.VectorSubcoreMesh(core_axis_name="c", subcore_axis_name="s")
_params = pltpu.CompilerParams(needs_layout_passes=False,
                               use_tc_tiling_on_sc=False)


def _bcast15(v):
    return lax.gather(
        v, jnp.full((16, 1), 15, jnp.int32),
        lax.GatherDimensionNumbers(offset_dims=(), collapsed_slice_dims=(0,),
                                   start_index_map=(0,)),
        slice_sizes=(1,), mode=lax.GatherScatterMode.PROMISE_IN_BOUNDS)


def _load_idx_2d(idx_hbm, base, n, idx1_v, idx2_v):
    pltpu.sync_copy(idx_hbm.at[pl.ds(base, n)], idx1_v)
    for i in range(n // 16):
        idx2_v[i // 8, pl.ds((i % 8) * 16, 16)] = idx1_v[pl.ds(i * 16, 16)]


@functools.partial(
    pl.kernel, mesh=_mesh,
    out_type=jax.ShapeDtypeStruct((B, D), jnp.float32),
    compiler_params=_params,
    scratch_types=[
        pltpu.VMEM((BPW,), jnp.int32),
        pltpu.VMEM((BPW // 128, 128), jnp.int32),
        pltpu.VMEM((BPW, D), jnp.float32),
    ],
)
def _sc_gather(mem_hbm, idx_hbm, old_hbm, idx1_v, idx2_v, rows_v):
    wid = lax.axis_index("s") * NC + lax.axis_index("c")
    _load_idx_2d(idx_hbm, wid * BPW, BPW, idx1_v, idx2_v)
    for j in range(BPW // 128):
        pltpu.sync_copy(mem_hbm.at[idx2_v.at[j]],
                        rows_v.at[pl.ds(j * 128, 128)])
    pltpu.sync_copy(rows_v, old_hbm.at[pl.ds(wid * BPW, BPW)])


@functools.partial(
    pl.kernel, mesh=_mesh,
    out_type=jax.ShapeDtypeStruct((M, D), jnp.float32),
    compiler_params=_params,
    scratch_types=[
        pltpu.VMEM_SHARED((BLK + SCRAP, D), jnp.float32),
        pltpu.VMEM((BPT,), jnp.int32),
        pltpu.VMEM((NCH, 128), jnp.int32),
        pltpu.VMEM((NCH, 128), jnp.int32),
        pltpu.VMEM((NCH, 128), jnp.int32),
        pltpu.VMEM((128, D), jnp.float32),
    ],
)
def _sc_scatter(mem_hbm, delta_hbm, idx_hbm, out_hbm,
                acc_sh, idx1_v, idx_v, selj_v, sell_v, buf_v):
    cid = lax.axis_index("c")
    sid = lax.axis_index("s")
    _load_idx_2d(idx_hbm, sid * BPT, BPT, idx1_v, idx_v)
    cs = jnp.minimum(sid * CHUNK, BLK - CHUNK)
    for b in range(NBLK // NC):
        blk = cid * (NBLK // NC) + b
        start = jnp.minimum(blk * BLK, M - BLK)
        pltpu.sync_copy(mem_hbm.at[pl.ds(start + cs, CHUNK)],
                        acc_sh.at[pl.ds(cs, CHUNK)])
        for i in range(BPT // 16):
            lane = lax.iota(jnp.int32, 16) + i * 16
            sl = pl.ds((i % 8) * 16, 16)
            selj_v[i // 8, sl] = (lane + sid * 64) & (B - 1)
            sell_v[i // 8, sl] = BLK + ((lane + sid * 32) & (SCRAP - 1))
        off = jnp.zeros((16,), jnp.int32)
        for i in range(BPT // 16):
            iv = idx_v[i // 8, pl.ds((i % 8) * 16, 16)]
            inb = (iv >= start) & (iv < start + BLK)
            inc = plsc.cumsum(jnp.where(inb, 1, 0))
            pos = jnp.maximum(off + inc - 1, 0)
            row = lax.shift_right_logical(pos, 7)
            col = pos & 127
            jrow = lax.iota(jnp.int32, 16) + (sid * BPT + i * 16)
            plsc.store_scatter(selj_v, [row, col], jrow, mask=inb)
            plsc.store_scatter(sell_v, [row, col], iv - start, mask=inb)
            off = off + _bcast15(inc)
        plsc.subcore_barrier()
        for c in range(NCH):
            @pl.when(jnp.any(off > c * 128))
            def _():
                pltpu.sync_copy(delta_hbm.at[selj_v.at[c]], buf_v)
                pltpu.sync_copy(buf_v, acc_sh.at[sell_v.at[c]], add=True)
        plsc.subcore_barrier()
        pltpu.sync_copy(acc_sh.at[pl.ds(cs, CHUNK)],
                        out_hbm.at[pl.ds(start + cs, CHUNK)])
        plsc.subcore_barrier()


def _delta_body(old_ref, val_ref, wg_ref, bg_ref, wv_ref, out_ref):
    old = old_ref[...]
    v = val_ref[...]
    pre = jnp.dot(old + v, wg_ref[...], preferred_element_type=jnp.float32)
    gate = jax.nn.sigmoid(pre + bg_ref[...])
    upd = jnp.tanh(jnp.dot(v, wv_ref[...], preferred_element_type=jnp.float32))
    out_ref[...] = gate * (upd - old)


_TCB = 4096


def _tc_delta(old, values, W_gate, b_gate, W_val):
    return pl.pallas_call(
        _delta_body,
        grid=(B // _TCB,),
        in_specs=[
            pl.BlockSpec((_TCB, D), lambda i: (i, 0)),
            pl.BlockSpec((_TCB, D), lambda i: (i, 0)),
            pl.BlockSpec((D, D), lambda i: (0, 0)),
            pl.BlockSpec((1, D), lambda i: (0, 0)),
            pl.BlockSpec((D, D), lambda i: (0, 0)),
        ],
        out_specs=pl.BlockSpec((_TCB, D), lambda i: (i, 0)),
        out_shape=jax.ShapeDtypeStruct((B, D), jnp.float32),
    )(old, values, W_gate, b_gate, W_val)


def kernel(memory, indices, values, W_gate, b_gate, W_val):
    idx = indices.astype(jnp.int32)
    old = _sc_gather(memory, idx)
    delta = _tc_delta(old, values, W_gate, b_gate.reshape(1, D), W_val)
    return _sc_scatter(memory, delta, idx)


# speedup vs baseline: 1.2198x; 1.0524x over previous
import functools

import jax
import jax.numpy as jnp
from jax import lax
from jax.experimental import pallas as pl
from jax.experimental.pallas import tpu as pltpu
from jax.experimental.pallas import tpu_sc as plsc

M = 100000
D = 64
B = 16384
NC, NS = 2, 16
NW = NC * NS
BPW = B // NW
BPT = B // NS
NBLK = 4
BLK = 25008
SCRAP = 512
CHUNK = 1568
NCH = BPT // 128

_mesh = plsc.VectorSubcoreMesh(core_axis_name="c", subcore_axis_name="s")
_params = pltpu.CompilerParams(needs_layout_passes=False,
                               use_tc_tiling_on_sc=False)


def _bcast15(v):
    return lax.gather(
        v, jnp.full((16, 1), 15, jnp.int32),
        lax.GatherDimensionNumbers(offset_dims=(), collapsed_slice_dims=(0,),
                                   start_index_map=(0,)),
        slice_sizes=(1,), mode=lax.GatherScatterMode.PROMISE_IN_BOUNDS)


def _load_idx_2d(idx_hbm, base, n, idx1_v, idx2_v):
    pltpu.sync_copy(idx_hbm.at[pl.ds(base, n)], idx1_v)
    for i in range(n // 16):
        idx2_v[i // 8, pl.ds((i % 8) * 16, 16)] = idx1_v[pl.ds(i * 16, 16)]


@functools.partial(
    pl.kernel, mesh=_mesh,
    out_type=jax.ShapeDtypeStruct((B, D), jnp.float32),
    compiler_params=_params,
    scratch_types=[
        pltpu.VMEM((BPW,), jnp.int32),
        pltpu.VMEM((BPW // 128, 128), jnp.int32),
        pltpu.VMEM((BPW, D), jnp.float32),
    ],
)
def _sc_gather(mem_hbm, idx_hbm, old_hbm, idx1_v, idx2_v, rows_v):
    wid = lax.axis_index("s") * NC + lax.axis_index("c")
    _load_idx_2d(idx_hbm, wid * BPW, BPW, idx1_v, idx2_v)
    for j in range(BPW // 128):
        pltpu.sync_copy(mem_hbm.at[idx2_v.at[j]],
                        rows_v.at[pl.ds(j * 128, 128)])
    pltpu.sync_copy(rows_v, old_hbm.at[pl.ds(wid * BPW, BPW)])


@functools.partial(
    pl.kernel, mesh=_mesh,
    out_type=jax.ShapeDtypeStruct((M, D), jnp.float32),
    compiler_params=_params,
    scratch_types=[
        pltpu.VMEM_SHARED((BLK + SCRAP, D), jnp.float32),
        pltpu.VMEM((BPT,), jnp.int32),
        pltpu.VMEM((NCH, 128), jnp.int32),
        pltpu.VMEM((NCH, 128), jnp.int32),
        pltpu.VMEM((NCH, 128), jnp.int32),
        pltpu.VMEM((128, D), jnp.float32),
    ],
)
def _sc_scatter(mem_hbm, delta_hbm, idx_hbm, out_hbm,
                acc_sh, idx1_v, idx_v, selj_v, sell_v, buf_v):
    cid = lax.axis_index("c")
    sid = lax.axis_index("s")
    _load_idx_2d(idx_hbm, sid * BPT, BPT, idx1_v, idx_v)
    cs = jnp.minimum(sid * CHUNK, BLK - CHUNK)
    for b in range(NBLK // NC):
        blk = cid * (NBLK // NC) + b
        start = jnp.minimum(blk * BLK, M - BLK)
        pltpu.sync_copy(mem_hbm.at[pl.ds(start + cs, CHUNK)],
                        acc_sh.at[pl.ds(cs, CHUNK)])
        for i in range(BPT // 16):
            lane = lax.iota(jnp.int32, 16) + i * 16
            sl = pl.ds((i % 8) * 16, 16)
            selj_v[i // 8, sl] = (lane + sid * 64) & (B - 1)
            sell_v[i // 8, sl] = BLK + ((lane + sid * 32) & (SCRAP - 1))
        off = jnp.zeros((16,), jnp.int32)
        for i in range(BPT // 16):
            iv = idx_v[i // 8, pl.ds((i % 8) * 16, 16)]
            inb = (iv >= start) & (iv < start + BLK)
            inc = plsc.cumsum(jnp.where(inb, 1, 0))
            pos = jnp.maximum(off + inc - 1, 0)
            row = lax.shift_right_logical(pos, 7)
            col = pos & 127
            jrow = lax.iota(jnp.int32, 16) + (sid * BPT + i * 16)
            plsc.store_scatter(selj_v, [row, col], jrow, mask=inb)
            plsc.store_scatter(sell_v, [row, col], iv - start, mask=inb)
            off = off + _bcast15(inc)
        plsc.subcore_barrier()
        for c in range(NCH):
            @pl.when(jnp.any(off > c * 128))
            def _():
                pltpu.sync_copy(delta_hbm.at[selj_v.at[c]], buf_v)
                pltpu.sync_copy(buf_v, acc_sh.at[sell_v.at[c]], add=True)
        plsc.subcore_barrier()
        pltpu.sync_copy(acc_sh.at[pl.ds(cs, CHUNK)],
                        out_hbm.at[pl.ds(start + cs, CHUNK)])
        plsc.subcore_barrier()


def _delta_body(old_ref, val_ref, wg_ref, bg_ref, wv_ref, out_ref):
    old = old_ref[...]
    v = val_ref[...]
    pre = jnp.dot(old + v, wg_ref[...], preferred_element_type=jnp.float32)
    gate = jax.nn.sigmoid(pre + bg_ref[...])
    upd = jnp.tanh(jnp.dot(v, wv_ref[...], preferred_element_type=jnp.float32))
    out_ref[...] = gate * (upd - old)


_TCB = 4096
_BP = B // 2


def _tc_delta(old_p, val_p, W2g, b2, W2v):
    return pl.pallas_call(
        _delta_body,
        grid=(_BP // _TCB,),
        in_specs=[
            pl.BlockSpec((_TCB, 2 * D), lambda i: (i, 0)),
            pl.BlockSpec((_TCB, 2 * D), lambda i: (i, 0)),
            pl.BlockSpec((2 * D, 2 * D), lambda i: (0, 0)),
            pl.BlockSpec((1, 2 * D), lambda i: (0, 0)),
            pl.BlockSpec((2 * D, 2 * D), lambda i: (0, 0)),
        ],
        out_specs=pl.BlockSpec((_TCB, 2 * D), lambda i: (i, 0)),
        out_shape=jax.ShapeDtypeStruct((_BP, 2 * D), jnp.float32),
    )(old_p, val_p, W2g, b2, W2v)


def _blockdiag2(w):
    z = jnp.zeros((D, D), w.dtype)
    return jnp.concatenate(
        [jnp.concatenate([w, z], 1), jnp.concatenate([z, w], 1)], 0)


def kernel(memory, indices, values, W_gate, b_gate, W_val):
    idx = indices.astype(jnp.int32)
    old = _sc_gather(memory, idx)
    delta_p = _tc_delta(old.reshape(_BP, 2 * D), values.reshape(_BP, 2 * D),
                        _blockdiag2(W_gate), jnp.tile(b_gate, 2).reshape(1, 2 * D),
                        _blockdiag2(W_val))
    return _sc_scatter(memory, delta_p.reshape(B, D), idx)


# speedup vs baseline: 1.2263x; 1.0053x over previous
import functools

import jax
import jax.numpy as jnp
from jax import lax
from jax.experimental import pallas as pl
from jax.experimental.pallas import tpu as pltpu
from jax.experimental.pallas import tpu_sc as plsc

M = 100000
D = 64
B = 16384
NC, NS = 2, 16
NW = NC * NS
BPW = B // NW
BPT = B // NS
NBLK = 8
BLK = 12504
SCRAP = 512
CHUNK = 784
NCH = BPT // 128

_mesh = plsc.VectorSubcoreMesh(core_axis_name="c", subcore_axis_name="s")
_params = pltpu.CompilerParams(needs_layout_passes=False,
                               use_tc_tiling_on_sc=False)


def _bcast15(v):
    return lax.gather(
        v, jnp.full((16, 1), 15, jnp.int32),
        lax.GatherDimensionNumbers(offset_dims=(), collapsed_slice_dims=(0,),
                                   start_index_map=(0,)),
        slice_sizes=(1,), mode=lax.GatherScatterMode.PROMISE_IN_BOUNDS)


def _load_idx_2d(idx_hbm, base, n, idx1_v, idx2_v):
    pltpu.sync_copy(idx_hbm.at[pl.ds(base, n)], idx1_v)
    for i in range(n // 16):
        idx2_v[i // 8, pl.ds((i % 8) * 16, 16)] = idx1_v[pl.ds(i * 16, 16)]


@functools.partial(
    pl.kernel, mesh=_mesh,
    out_type=jax.ShapeDtypeStruct((B, D), jnp.float32),
    compiler_params=_params,
    scratch_types=[
        pltpu.VMEM((BPW,), jnp.int32),
        pltpu.VMEM((BPW // 128, 128), jnp.int32),
        pltpu.VMEM((BPW, D), jnp.float32),
    ],
)
def _sc_gather(mem_hbm, idx_hbm, old_hbm, idx1_v, idx2_v, rows_v):
    wid = lax.axis_index("s") * NC + lax.axis_index("c")
    _load_idx_2d(idx_hbm, wid * BPW, BPW, idx1_v, idx2_v)
    for j in range(BPW // 128):
        pltpu.sync_copy(mem_hbm.at[idx2_v.at[j]],
                        rows_v.at[pl.ds(j * 128, 128)])
    pltpu.sync_copy(rows_v, old_hbm.at[pl.ds(wid * BPW, BPW)])


@functools.partial(
    pl.kernel, mesh=_mesh,
    out_type=jax.ShapeDtypeStruct((M, D), jnp.float32),
    compiler_params=_params,
    scratch_types=[
        pltpu.VMEM_SHARED((BLK + SCRAP, D), jnp.float32),
        pltpu.VMEM_SHARED((BLK + SCRAP, D), jnp.float32),
        pltpu.VMEM((BPT,), jnp.int32),
        pltpu.VMEM((NCH, 128), jnp.int32),
        pltpu.VMEM((NCH, 128), jnp.int32),
        pltpu.VMEM((NCH, 128), jnp.int32),
        pltpu.VMEM((128, D), jnp.float32),
        pltpu.SemaphoreType.DMA,
        pltpu.SemaphoreType.DMA,
    ],
)
def _sc_scatter(mem_hbm, delta_hbm, idx_hbm, out_hbm,
                accA_sh, accB_sh, idx1_v, idx_v, selj_v, sell_v, buf_v,
                semA, semB):
    cid = lax.axis_index("c")
    sid = lax.axis_index("s")
    _load_idx_2d(idx_hbm, sid * BPT, BPT, idx1_v, idx_v)
    cs = jnp.minimum(sid * CHUNK, BLK - CHUNK)
    outcopy = [None, None]
    for b in range(NBLK // NC):
        acc_sh = accA_sh if b % 2 == 0 else accB_sh
        sem = semA if b % 2 == 0 else semB
        blk = cid * (NBLK // NC) + b
        start = jnp.minimum(blk * BLK, M - BLK)
        if outcopy[b % 2] is not None:
            outcopy[b % 2].wait()
        pltpu.sync_copy(mem_hbm.at[pl.ds(start + cs, CHUNK)],
                        acc_sh.at[pl.ds(cs, CHUNK)])
        for i in range(BPT // 16):
            lane = lax.iota(jnp.int32, 16) + i * 16
            sl = pl.ds((i % 8) * 16, 16)
            selj_v[i // 8, sl] = (lane + sid * 64) & (B - 1)
            sell_v[i // 8, sl] = BLK + ((lane + sid * 32) & (SCRAP - 1))
        off = jnp.zeros((16,), jnp.int32)
        for i in range(BPT // 16):
            iv = idx_v[i // 8, pl.ds((i % 8) * 16, 16)]
            inb = (iv >= start) & (iv < start + BLK)
            inc = plsc.cumsum(jnp.where(inb, 1, 0))
            pos = jnp.maximum(off + inc - 1, 0)
            row = lax.shift_right_logical(pos, 7)
            col = pos & 127
            jrow = lax.iota(jnp.int32, 16) + (sid * BPT + i * 16)
            plsc.store_scatter(selj_v, [row, col], jrow, mask=inb)
            plsc.store_scatter(sell_v, [row, col], iv - start, mask=inb)
            off = off + _bcast15(inc)
        plsc.subcore_barrier()
        for c in range(NCH):
            @pl.when(jnp.any(off > c * 128))
            def _():
                pltpu.sync_copy(delta_hbm.at[selj_v.at[c]], buf_v)
                pltpu.sync_copy(buf_v, acc_sh.at[sell_v.at[c]], add=True)
        plsc.subcore_barrier()
        outcopy[b % 2] = pltpu.async_copy(
            acc_sh.at[pl.ds(cs, CHUNK)],
            out_hbm.at[pl.ds(start + cs, CHUNK)], sem)
    for h in outcopy:
        if h is not None:
            h.wait()


def _delta_body(old_ref, val_ref, wg_ref, bg_ref, wv_ref, out_ref):
    old = old_ref[...]
    v = val_ref[...]
    pre = jnp.dot(old + v, wg_ref[...], preferred_element_type=jnp.float32)
    gate = jax.nn.sigmoid(pre + bg_ref[...])
    upd = jnp.tanh(jnp.dot(v, wv_ref[...], preferred_element_type=jnp.float32))
    out_ref[...] = gate * (upd - old)


_TCB = 4096
_BP = B // 2


def _tc_delta(old_p, val_p, W2g, b2, W2v):
    return pl.pallas_call(
        _delta_body,
        grid=(_BP // _TCB,),
        in_specs=[
            pl.BlockSpec((_TCB, 2 * D), lambda i: (i, 0)),
            pl.BlockSpec((_TCB, 2 * D), lambda i: (i, 0)),
            pl.BlockSpec((2 * D, 2 * D), lambda i: (0, 0)),
            pl.BlockSpec((1, 2 * D), lambda i: (0, 0)),
            pl.BlockSpec((2 * D, 2 * D), lambda i: (0, 0)),
        ],
        out_specs=pl.BlockSpec((_TCB, 2 * D), lambda i: (i, 0)),
        out_shape=jax.ShapeDtypeStruct((_BP, 2 * D), jnp.float32),
    )(old_p, val_p, W2g, b2, W2v)


def _blockdiag2(w):
    z = jnp.zeros((D, D), w.dtype)
    return jnp.concatenate(
        [jnp.concatenate([w, z], 1), jnp.concatenate([z, w], 1)], 0)


def kernel(memory, indices, values, W_gate, b_gate, W_val):
    idx = indices.astype(jnp.int32)
    old = _sc_gather(memory, idx)
    delta_p = _tc_delta(old.reshape(_BP, 2 * D), values.reshape(_BP, 2 * D),
                        _blockdiag2(W_gate), jnp.tile(b_gate, 2).reshape(1, 2 * D),
                        _blockdiag2(W_val))
    return _sc_scatter(memory, delta_p.reshape(B, D), idx)


# speedup vs baseline: 1.2564x; 1.0246x over previous
import functools

import jax
import jax.numpy as jnp
from jax import lax
from jax.experimental import pallas as pl
from jax.experimental.pallas import tpu as pltpu
from jax.experimental.pallas import tpu_sc as plsc

M = 100000
D = 64
B = 16384
NC, NS = 2, 16
NW = NC * NS
BPW = B // NW
BPT = B // NS
NBLK = 8
BLK = 12504
SCRAP = 512
CHUNK = 784
NCH = BPT // 128

_mesh = plsc.VectorSubcoreMesh(core_axis_name="c", subcore_axis_name="s")
_params = pltpu.CompilerParams(needs_layout_passes=False,
                               use_tc_tiling_on_sc=False)


def _bcast15(v):
    return lax.gather(
        v, jnp.full((16, 1), 15, jnp.int32),
        lax.GatherDimensionNumbers(offset_dims=(), collapsed_slice_dims=(0,),
                                   start_index_map=(0,)),
        slice_sizes=(1,), mode=lax.GatherScatterMode.PROMISE_IN_BOUNDS)


def _load_idx_2d(idx_hbm, base, n, idx1_v, idx2_v):
    pltpu.sync_copy(idx_hbm.at[pl.ds(base, n)], idx1_v)
    for i in range(n // 16):
        idx2_v[i // 8, pl.ds((i % 8) * 16, 16)] = idx1_v[pl.ds(i * 16, 16)]


@functools.partial(
    pl.kernel, mesh=_mesh,
    out_type=jax.ShapeDtypeStruct((B, D), jnp.float32),
    compiler_params=_params,
    scratch_types=[
        pltpu.VMEM((BPW,), jnp.int32),
        pltpu.VMEM((BPW // 128, 128), jnp.int32),
        pltpu.VMEM((BPW, D), jnp.float32),
    ],
)
def _sc_gather(mem_hbm, idx_hbm, old_hbm, idx1_v, idx2_v, rows_v):
    wid = lax.axis_index("s") * NC + lax.axis_index("c")
    _load_idx_2d(idx_hbm, wid * BPW, BPW, idx1_v, idx2_v)
    for j in range(BPW // 128):
        pltpu.sync_copy(mem_hbm.at[idx2_v.at[j]],
                        rows_v.at[pl.ds(j * 128, 128)])
    pltpu.sync_copy(rows_v, old_hbm.at[pl.ds(wid * BPW, BPW)])


@functools.partial(
    pl.kernel, mesh=_mesh,
    out_type=jax.ShapeDtypeStruct((M, D), jnp.float32),
    compiler_params=_params,
    scratch_types=[
        pltpu.VMEM_SHARED((BLK + SCRAP, D), jnp.float32),
        pltpu.VMEM_SHARED((BLK + SCRAP, D), jnp.float32),
        pltpu.VMEM((BPT,), jnp.int32),
        pltpu.VMEM((NCH, 128), jnp.int32),
        pltpu.VMEM((NCH, 128), jnp.int32),
        pltpu.VMEM((NCH, 128), jnp.int32),
        pltpu.VMEM((128, D), jnp.float32),
        pltpu.SemaphoreType.DMA,
        pltpu.SemaphoreType.DMA,
        pltpu.SemaphoreType.DMA,
    ],
)
def _sc_scatter(mem_hbm, delta_hbm, idx_hbm, out_hbm,
                accA_sh, accB_sh, idx1_v, idx_v, selj_v, sell_v, buf_v,
                semA, semB, semC):
    cid = lax.axis_index("c")
    sid = lax.axis_index("s")
    _load_idx_2d(idx_hbm, sid * BPT, BPT, idx1_v, idx_v)
    cs = jnp.minimum(sid * CHUNK, BLK - CHUNK)
    outcopy = [None, None]
    for b in range(NBLK // NC):
        acc_sh = accA_sh if b % 2 == 0 else accB_sh
        sem = semA if b % 2 == 0 else semB
        blk = cid * (NBLK // NC) + b
        start = jnp.minimum(blk * BLK, M - BLK)
        if outcopy[b % 2] is not None:
            outcopy[b % 2].wait()
        incopy = pltpu.async_copy(mem_hbm.at[pl.ds(start + cs, CHUNK)],
                                  acc_sh.at[pl.ds(cs, CHUNK)], semC)
        for i in range(BPT // 16):
            lane = lax.iota(jnp.int32, 16) + i * 16
            sl = pl.ds((i % 8) * 16, 16)
            selj_v[i // 8, sl] = (lane + sid * 64) & (B - 1)
            sell_v[i // 8, sl] = BLK + ((lane + sid * 32) & (SCRAP - 1))
        off = jnp.zeros((16,), jnp.int32)
        for i in range(BPT // 16):
            iv = idx_v[i // 8, pl.ds((i % 8) * 16, 16)]
            inb = (iv >= start) & (iv < start + BLK)
            inc = plsc.cumsum(jnp.where(inb, 1, 0))
            pos = jnp.maximum(off + inc - 1, 0)
            row = lax.shift_right_logical(pos, 7)
            col = pos & 127
            jrow = lax.iota(jnp.int32, 16) + (sid * BPT + i * 16)
            plsc.store_scatter(selj_v, [row, col], jrow, mask=inb)
            plsc.store_scatter(sell_v, [row, col], iv - start, mask=inb)
            off = off + _bcast15(inc)
        incopy.wait()
        plsc.subcore_barrier()
        for c in range(NCH):
            @pl.when(jnp.any(off > c * 128))
            def _():
                pltpu.sync_copy(delta_hbm.at[selj_v.at[c]], buf_v)
                pltpu.sync_copy(buf_v, acc_sh.at[sell_v.at[c]], add=True)
        plsc.subcore_barrier()
        outcopy[b % 2] = pltpu.async_copy(
            acc_sh.at[pl.ds(cs, CHUNK)],
            out_hbm.at[pl.ds(start + cs, CHUNK)], sem)
    for h in outcopy:
        if h is not None:
            h.wait()


def _delta_body(old_ref, val_ref, wg_ref, bg_ref, wv_ref, out_ref):
    old = old_ref[...]
    v = val_ref[...]
    pre = jnp.dot(old + v, wg_ref[...], preferred_element_type=jnp.float32)
    gate = jax.nn.sigmoid(pre + bg_ref[...])
    upd = jnp.tanh(jnp.dot(v, wv_ref[...], preferred_element_type=jnp.float32))
    out_ref[...] = gate * (upd - old)


_TCB = 4096
_BP = B // 2


def _tc_delta(old_p, val_p, W2g, b2, W2v):
    return pl.pallas_call(
        _delta_body,
        grid=(_BP // _TCB,),
        in_specs=[
            pl.BlockSpec((_TCB, 2 * D), lambda i: (i, 0)),
            pl.BlockSpec((_TCB, 2 * D), lambda i: (i, 0)),
            pl.BlockSpec((2 * D, 2 * D), lambda i: (0, 0)),
            pl.BlockSpec((1, 2 * D), lambda i: (0, 0)),
            pl.BlockSpec((2 * D, 2 * D), lambda i: (0, 0)),
        ],
        out_specs=pl.BlockSpec((_TCB, 2 * D), lambda i: (i, 0)),
        out_shape=jax.ShapeDtypeStruct((_BP, 2 * D), jnp.float32),
    )(old_p, val_p, W2g, b2, W2v)


def _blockdiag2(w):
    z = jnp.zeros((D, D), w.dtype)
    return jnp.concatenate(
        [jnp.concatenate([w, z], 1), jnp.concatenate([z, w], 1)], 0)


def kernel(memory, indices, values, W_gate, b_gate, W_val):
    idx = indices.astype(jnp.int32)
    old = _sc_gather(memory, idx)
    delta_p = _tc_delta(old.reshape(_BP, 2 * D), values.reshape(_BP, 2 * D),
                        _blockdiag2(W_gate), jnp.tile(b_gate, 2).reshape(1, 2 * D),
                        _blockdiag2(W_val))
    return _sc_scatter(memory, delta_p.reshape(B, D), idx)
